# baseline (speedup 1.0000x reference)
import math

import jax
import jax.numpy as jnp
from jax import lax
from jax.experimental import pallas as pl
from jax.experimental.pallas import tpu as pltpu
from jax.experimental.pallas import tpu_sc as plsc

_D = 1024
_L = 2048
_B = 4
_NC = 2
_NS = 16
_NW = _NC * _NS
_PE_ROWS = _L // _NW
_SEG_ROWS = 32
_NSEG = _PE_ROWS // _SEG_ROWS
_CHUNK_ROWS = 16
_CHUNK_ELEMS = _CHUNK_ROWS * _D
_KPS = _SEG_ROWS // _CHUNK_ROWS
_NCHUNKS = _NSEG * _B * _KPS
_LANES = 16
_CPR = _D // _LANES


def _sc_body(x_hbm, pe_hbm, out_hbm, pe_buf, ib, ob, s_pe,
             s_in0, s_in1, s_out0, s_out1):
    inv_scale = 1.0 / math.sqrt(_D)
    in_sems = (s_in0, s_in1)
    out_sems = (s_out0, s_out1)
    wid = lax.axis_index("s") * _NC + lax.axis_index("c")
    row0 = wid * _PE_ROWS

    def decomp(j):
        h, r = divmod(j, _B * _KPS)
        b, k = divmod(r, _KPS)
        return h, b, k

    def x_slice(j):
        h, b, k = decomp(j)
        rows = row0 + h * _SEG_ROWS + k * _CHUNK_ROWS
        return (b, pl.ds(rows, _CHUNK_ROWS), slice(None))

    def start_in(j, p):
        pltpu.async_copy(x_hbm.at[x_slice(j)], ib.at[p], in_sems[p])

    def wait_in(j, p):
        pltpu.make_async_copy(x_hbm.at[x_slice(j)], ib.at[p], in_sems[p]).wait()

    def start_out(j, p):
        pltpu.async_copy(ob.at[p], out_hbm.at[x_slice(j)], out_sems[p])

    def wait_out(j, p):
        pltpu.make_async_copy(ob.at[p], out_hbm.at[x_slice(j)], out_sems[p]).wait()

    def pe_seg_src(h):
        return pe_hbm.at[pl.ds(row0 + h * _SEG_ROWS, _SEG_ROWS), :]

    pltpu.async_copy(pe_seg_src(0), pe_buf, s_pe)
    start_in(0, 0)
    start_in(1, 1)
    pltpu.make_async_copy(pe_seg_src(0), pe_buf, s_pe).wait()

    for j in range(_NCHUNKS):
        p = j % 2
        if j == _NCHUNKS // 2:
            pltpu.sync_copy(pe_seg_src(1), pe_buf)
        wait_in(j, p)
        if j >= 2:
            wait_out(j - 2, p)
        k = decomp(j)[2]

        @plsc.parallel_loop(0, _CHUNK_ELEMS // _LANES, unroll=8)
        def _add(i, p=p, k=k):
            r = i // _CPR
            sl = pl.ds((i % _CPR) * _LANES, _LANES)
            ob[p, r, sl] = ib[p, r, sl] + pe_buf[k * _CHUNK_ROWS + r, sl] * inv_scale

        start_out(j, p)
        if j + 2 < _NCHUNKS:
            start_in(j + 2, p)

    wait_out(_NCHUNKS - 2, 0)
    wait_out(_NCHUNKS - 1, 1)


def kernel(x, pe):
    b, l, d = x.shape
    mesh = plsc.VectorSubcoreMesh(core_axis_name="c", subcore_axis_name="s")
    fn = pl.kernel(
        _sc_body,
        out_type=jax.ShapeDtypeStruct((b, l, d), x.dtype),
        mesh=mesh,
        scratch_types=[
            pltpu.VMEM((_SEG_ROWS, _D), jnp.float32),
            pltpu.VMEM((2, _CHUNK_ROWS, _D), jnp.float32),
            pltpu.VMEM((2, _CHUNK_ROWS, _D), jnp.float32),
            pltpu.SemaphoreType.DMA,
            pltpu.SemaphoreType.DMA,
            pltpu.SemaphoreType.DMA,
            pltpu.SemaphoreType.DMA,
            pltpu.SemaphoreType.DMA,
        ],
        compiler_params=pltpu.CompilerParams(use_tc_tiling_on_sc=True),
    )
    return fn(x, pe[:l])


# speedup vs baseline: 1.0087x; 1.0087x over previous
import math

import jax
import jax.numpy as jnp
from jax import lax
from jax.experimental import pallas as pl
from jax.experimental.pallas import tpu as pltpu
from jax.experimental.pallas import tpu_sc as plsc

_D = 1024
_L = 2048
_B = 4
_NC = 2
_NS = 16
_NW = _NC * _NS
_PE_ROWS = _L // _NW
_CHUNK_ROWS = 16
_CHUNK_ELEMS = _CHUNK_ROWS * _D
_KPB = _PE_ROWS // _CHUNK_ROWS
_NCHUNKS = _B * _KPB
_NBUF = 3
_LANES = 16
_CPR = _D // _LANES


def _sc_body(x_hbm, pe_hbm, out_hbm, pe_buf, xb, s_pe, s_in0, s_in1, s_in2,
             s_out0, s_out1, s_out2):
    inv_scale = 1.0 / math.sqrt(_D)
    in_sems = (s_in0, s_in1, s_in2)
    out_sems = (s_out0, s_out1, s_out2)
    wid = lax.axis_index("s") * _NC + lax.axis_index("c")
    row0 = wid * _PE_ROWS

    def x_slice(j):
        b, k = divmod(j, _KPB)
        return (b, pl.ds(row0 + k * _CHUNK_ROWS, _CHUNK_ROWS), slice(None))

    def start_in(j, p):
        pltpu.async_copy(x_hbm.at[x_slice(j)], xb.at[p], in_sems[p])

    def wait_in(j, p):
        pltpu.make_async_copy(x_hbm.at[x_slice(j)], xb.at[p], in_sems[p]).wait()

    def start_out(j, p):
        pltpu.async_copy(xb.at[p], out_hbm.at[x_slice(j)], out_sems[p])

    def wait_out(j, p):
        pltpu.make_async_copy(xb.at[p], out_hbm.at[x_slice(j)], out_sems[p]).wait()

    pe_src = pe_hbm.at[pl.ds(row0, _PE_ROWS), :]
    pltpu.async_copy(pe_src, pe_buf, s_pe)
    for j in range(_NBUF):
        start_in(j, j)
    pltpu.make_async_copy(pe_src, pe_buf, s_pe).wait()

    for j in range(_NCHUNKS):
        p = j % _NBUF
        if _NBUF - 1 <= j <= _NCHUNKS - 2:
            wait_out(j + 1 - _NBUF, (j + 1) % _NBUF)
            start_in(j + 1, (j + 1) % _NBUF)
        wait_in(j, p)
        k = j % _KPB

        @plsc.parallel_loop(0, _CHUNK_ELEMS // _LANES, unroll=8)
        def _add(i, p=p, k=k):
            r = i // _CPR
            sl = pl.ds((i % _CPR) * _LANES, _LANES)
            xb[p, r, sl] = xb[p, r, sl] + pe_buf[k * _CHUNK_ROWS + r, sl] * inv_scale

        start_out(j, p)

    for j in range(_NCHUNKS - _NBUF, _NCHUNKS):
        wait_out(j, j % _NBUF)


def kernel(x, pe):
    b, l, d = x.shape
    mesh = plsc.VectorSubcoreMesh(core_axis_name="c", subcore_axis_name="s")
    fn = pl.kernel(
        _sc_body,
        out_type=jax.ShapeDtypeStruct((b, l, d), x.dtype),
        mesh=mesh,
        scratch_types=[
            pltpu.VMEM((_PE_ROWS, _D), jnp.float32),
            pltpu.VMEM((_NBUF, _CHUNK_ROWS, _D), jnp.float32),
            pltpu.SemaphoreType.DMA,
            pltpu.SemaphoreType.DMA,
            pltpu.SemaphoreType.DMA,
            pltpu.SemaphoreType.DMA,
            pltpu.SemaphoreType.DMA,
            pltpu.SemaphoreType.DMA,
            pltpu.SemaphoreType.DMA,
        ],
        compiler_params=pltpu.CompilerParams(use_tc_tiling_on_sc=True),
    )
    return fn(x, pe[:l])
